# Initial kernel scaffold; baseline (speedup 1.0000x reference)
import jax
import jax.numpy as jnp
from jax import lax
from jax.experimental import pallas as pl
from jax.experimental.pallas import tpu as pltpu

_RADIUS = 0.1
_RADIUS2 = _RADIUS * _RADIUS
_K = 2
_TM = 256


def _bq_body(x_ref, h_ref, o_ref):
    c = x_ref[0]
    p = h_ref[0]
    n = p.shape[1]
    tm = c.shape[1]

    c2 = (c[0] * c[0] + c[1] * c[1]) + c[2] * c[2]
    p2 = (p[0] * p[0] + p[1] * p[1]) + p[2] * p[2]
    dot = (c[0][:, None] * p[0][None, :]
           + c[1][:, None] * p[1][None, :]) \
        + c[2][:, None] * p[2][None, :]
    dist2 = (c2[:, None] + p2[None, :]) - 2.0 * dot
    mask = dist2 < _RADIUS2

    sent = jnp.int32(n)
    iota = lax.broadcasted_iota(jnp.int32, (tm, n), 1)
    midx = jnp.where(mask, iota, sent)
    first = jnp.min(midx, axis=1)
    midx2 = jnp.where(midx == first[:, None], sent, midx)
    second = jnp.min(midx2, axis=1)

    out0 = jnp.where(first == sent, 0, first)
    out1 = jnp.where(second == sent, out0, second)
    o_ref[0] = jnp.stack([out0, out1], axis=0)


def kernel(x, h):
    b, _, m = x.shape
    n = h.shape[2]
    grid = (b, m // _TM)
    out = pl.pallas_call(
        _bq_body,
        grid=grid,
        in_specs=[
            pl.BlockSpec((1, 3, _TM), lambda i, j: (i, 0, j)),
            pl.BlockSpec((1, 3, n), lambda i, j: (i, 0, 0)),
        ],
        out_specs=pl.BlockSpec((1, _K, _TM), lambda i, j: (i, 0, j)),
        out_shape=jax.ShapeDtypeStruct((b, _K, m), jnp.int32),
    )(x, h)
    return jnp.transpose(out, (0, 2, 1))


# speedup vs baseline: 15.2508x; 15.2508x over previous
import jax
import jax.numpy as jnp
from jax import lax
from jax.experimental import pallas as pl
from jax.experimental.pallas import tpu as pltpu

_RADIUS = 0.1
_RADIUS2 = _RADIUS * _RADIUS
_K = 2
_TM = 256


def _bq_body(x_ref, h_ref, o_ref):
    c = x_ref[0]
    p = h_ref[0]
    n = p.shape[1]
    tm = c.shape[1]

    c2 = (c[0] * c[0] + c[1] * c[1]) + c[2] * c[2]
    p2 = (p[0] * p[0] + p[1] * p[1]) + p[2] * p[2]
    cb = c.astype(jnp.bfloat16)
    pb = p.astype(jnp.bfloat16)
    dot = lax.dot_general(cb, pb, (((0,), (0,)), ((), ())),
                          preferred_element_type=jnp.float32)
    dist2 = (c2[:, None] + p2[None, :]) - 2.0 * dot
    mask = dist2 < _RADIUS2

    sent = jnp.int32(n)
    iota = lax.broadcasted_iota(jnp.int32, (tm, n), 1)
    midx = jnp.where(mask, iota, sent)
    first = jnp.min(midx, axis=1)
    midx2 = jnp.where(midx == first[:, None], sent, midx)
    second = jnp.min(midx2, axis=1)

    out0 = jnp.where(first == sent, 0, first)
    out1 = jnp.where(second == sent, out0, second)
    o_ref[0] = jnp.stack([out0, out1], axis=0)


def kernel(x, h):
    b, _, m = x.shape
    n = h.shape[2]
    grid = (b, m // _TM)
    out = pl.pallas_call(
        _bq_body,
        grid=grid,
        in_specs=[
            pl.BlockSpec((1, 3, _TM), lambda i, j: (i, 0, j)),
            pl.BlockSpec((1, 3, n), lambda i, j: (i, 0, 0)),
        ],
        out_specs=pl.BlockSpec((1, _K, _TM), lambda i, j: (i, 0, j)),
        out_shape=jax.ShapeDtypeStruct((b, _K, m), jnp.int32),
    )(x, h)
    return jnp.transpose(out, (0, 2, 1))
